# Initial kernel scaffold; baseline (speedup 1.0000x reference)
import jax
import jax.numpy as jnp
from jax.experimental import pallas as pl


def kernel(edge_index, batch, W1a, b1a, W1b, b1b, W2a, b2a, W2b, b2b, W3a, b3a, W3b, b3b, W4a, b4a, W4b, b4b, Wd1, bd1, Wd2, bd2, w_pool):
    raise NotImplementedError("write your pallas kernel here")


# speedup vs baseline: 11.6881x; 11.6881x over previous
import functools

import jax
import jax.numpy as jnp
from jax import lax
from jax.experimental import pallas as pl
from jax.experimental.pallas import tpu as pltpu
from jax.experimental.pallas import tpu_sc as plsc

_N = 10000
_E = 320000
_G = 64
_H = 128
_OUT = 128

_NC = 2
_NS = 16
_NW = _NC * _NS
_CH = 128
_NCHUNK = _E // _CH
_RB = 200
_NBLK = _N // _RB


def _seg_sum_body(width, gather, x_hbm, src_hbm, dst_hbm, zero_hbm, out_hbm,
                  src_v, dst_v, rows_v, acc, sem):
    cid = lax.axis_index("c")
    sid = lax.axis_index("s")
    wid = sid * _NC + cid

    if True:
        nb = (_NBLK - sid + _NS - 1) // _NS

        def init_step(k, carry):
            base = (sid + k * _NS) * _RB
            pltpu.sync_copy(zero_hbm.at[pl.ds(base, _RB)],
                            acc.at[pl.ds(base, _RB)])
            return carry

        lax.fori_loop(0, nb, init_step, 0)
        if not gather:
            pltpu.sync_copy(x_hbm.at[pl.ds(0, _CH)], rows_v)
        plsc.subcore_barrier()

        nt = (_NCHUNK - wid + _NW - 1) // _NW

        def step(t, carry):
            base = (wid + t * _NW) * _CH
            pltpu.sync_copy(dst_hbm.at[pl.ds(base, _CH)], dst_v)
            if gather:
                pltpu.sync_copy(src_hbm.at[pl.ds(base, _CH)], src_v)
                pltpu.async_copy(x_hbm.at[src_v], rows_v, sem).wait()
            pltpu.sync_copy(rows_v, acc.at[dst_v], add=True)
            return carry

        lax.fori_loop(0, nt, step, 0)
        plsc.subcore_barrier()

        def wb_step(k, carry):
            base = (sid + k * _NS) * _RB
            pltpu.sync_copy(acc.at[pl.ds(base, _RB)],
                            out_hbm.at[cid, pl.ds(base, _RB)])
            return carry

        lax.fori_loop(0, nb, wb_step, 0)


def _make_seg_sum(width, gather):
    mesh = plsc.VectorSubcoreMesh(core_axis_name="c", subcore_axis_name="s")
    return pl.kernel(
        functools.partial(_seg_sum_body, width, gather),
        out_type=jax.ShapeDtypeStruct((_NC, _N, width), jnp.float32),
        mesh=mesh,
        scratch_types=[
            pltpu.VMEM((_CH,), jnp.int32),
            pltpu.VMEM((_CH,), jnp.int32),
            pltpu.VMEM((_CH, width), jnp.float32),
            pltpu.VMEM_SHARED((_N, width), jnp.float32),
            pltpu.SemaphoreType.DMA,
        ],
        name=f"sc_seg_sum_{width}_{int(gather)}",
    )


def _relu(x):
    return jnp.maximum(x, 0.0)


def _mm(a, b):
    return jax.lax.dot_general(a, b, (((1,), (0,)), ((), ())),
                               preferred_element_type=jnp.float32)


def _tc1_body(degp_ref, w1a_ref, b1a_ref, w1b_ref, b1b_ref, x1_ref):
    deg = degp_ref[0][:, :1] + degp_ref[1][:, :1]
    t = deg + 1.0
    h = _relu(t * w1a_ref[...] + b1a_ref[...])
    x1_ref[...] = _relu(_mm(h, w1b_ref[...]) + b1b_ref[...])


def _tc2_body(x1_ref, p_ref, batch_ref, wp_ref,
              w2a_ref, b2a_ref, w2b_ref, b2b_ref, x2m_ref, km_ref):
    h0 = x1_ref[...] + p_ref[0] + p_ref[1]
    h1 = _relu(_mm(h0, w2a_ref[...]) + b2a_ref[...])
    x2 = _relu(_mm(h1, w2b_ref[...]) + b2b_ref[...])
    s = jnp.sum(x2, axis=1, keepdims=True)
    wp = wp_ref[0, 0]
    score = jnp.tanh(s * (wp / jnp.abs(wp)))

    bat = batch_ref[...]
    gid = lax.broadcasted_iota(jnp.int32, (1, _G), 1)
    onehot = bat == gid
    counts = jnp.sum(onehot.astype(jnp.float32), axis=0, keepdims=True)
    kk = jnp.ceil(0.5 * counts)

    key = bat.astype(jnp.float32) * 4.0 - score
    b = jax.lax.bitcast_convert_type(key, jnp.uint32)
    neg = (b >> jnp.uint32(31)) > jnp.uint32(0)
    u = jnp.where(neg, ~b, b | jnp.uint32(0x80000000))
    v = ~u

    def bs_step(i, thr):
        bit = jnp.uint32(1) << (jnp.uint32(31) - i.astype(jnp.uint32))
        cand = thr | bit
        ge = jnp.logical_and(v >= cand, onehot)
        cnt = jnp.sum(ge.astype(jnp.float32), axis=0, keepdims=True)
        return jnp.where(cnt >= kk, cand, thr)

    thr = lax.fori_loop(0, 32, bs_step, jnp.zeros((1, _G), jnp.uint32))
    gt_ng = jnp.logical_and(v > thr, onehot)
    cnt_gt = jnp.sum(gt_ng.astype(jnp.float32), axis=0, keepdims=True)
    rr = kk - cnt_gt
    bnd = jnp.logical_and(v == thr, onehot)
    idx = lax.broadcasted_iota(jnp.uint32, (_N, 1), 0)
    w = jnp.uint32(_N) - idx

    def bs_step2(i, thr2):
        bit = jnp.uint32(1) << (jnp.uint32(15) - i.astype(jnp.uint32))
        cand = thr2 | bit
        ge = jnp.logical_and(w >= cand, bnd)
        cnt = jnp.sum(ge.astype(jnp.float32), axis=0, keepdims=True)
        return jnp.where(cnt >= rr, cand, thr2)

    thr2 = lax.fori_loop(0, 16, bs_step2, jnp.zeros((1, _G), jnp.uint32))
    keep_ng = jnp.logical_or(gt_ng, jnp.logical_and(bnd, w >= thr2))
    km = jnp.sum(keep_ng.astype(jnp.float32), axis=1, keepdims=True)
    km_ref[...] = km
    x2m_ref[...] = x2 * (score * km)


def _tc3_body(xm_ref, p_ref, km_ref, wa_ref, ba_ref, wb_ref, bb_ref, o_ref):
    h0 = xm_ref[...] + p_ref[0] + p_ref[1]
    h1 = _relu(_mm(h0, wa_ref[...]) + ba_ref[...])
    o_ref[...] = _relu(_mm(h1, wb_ref[...]) + bb_ref[...]) * km_ref[...]


def _tc4_body(xm_ref, p_ref, km_ref, batch_ref,
              w4a_ref, b4a_ref, w4b_ref, b4b_ref,
              wd1_ref, bd1_ref, wd2_ref, bd2_ref, out_ref):
    h0 = xm_ref[...] + p_ref[0] + p_ref[1]
    h1 = _relu(_mm(h0, w4a_ref[...]) + b4a_ref[...])
    x4 = _relu(_mm(h1, w4b_ref[...]) + b4b_ref[...]) * km_ref[...]
    gid = lax.broadcasted_iota(jnp.int32, (1, _G), 1)
    onehot = (batch_ref[...] == gid).astype(jnp.float32)
    pooled = jax.lax.dot_general(onehot, x4, (((0,), (0,)), ((), ())),
                                 preferred_element_type=jnp.float32)
    hh = _relu(_mm(pooled, wd1_ref[...]) + bd1_ref[...])
    out_ref[...] = _mm(hh, wd2_ref[...]) + bd2_ref[...]


def _tc_call(body, out_shape, *args):
    return pl.pallas_call(
        body, out_shape=out_shape, name=body.__name__)(*args)


def kernel(edge_index, batch, W1a, b1a, W1b, b1b, W2a, b2a, W2b, b2b,
           W3a, b3a, W3b, b3b, W4a, b4a, W4b, b4b, Wd1, bd1, Wd2, bd2,
           w_pool):
    src = edge_index[0]
    dst = edge_index[1]
    batch2 = batch.reshape(_N, 1)
    zeros_h = jnp.zeros((_N, _H), jnp.float32)
    ones_h = jnp.ones((_CH, _H), jnp.float32)

    deg_sum = _make_seg_sum(_H, gather=False)
    seg_sum = _make_seg_sum(_H, gather=True)

    b1a2, b1b2 = b1a.reshape(1, _H), b1b.reshape(1, _H)
    b2a2, b2b2 = b2a.reshape(1, _H), b2b.reshape(1, _H)
    b3a2, b3b2 = b3a.reshape(1, _H), b3b.reshape(1, _H)
    b4a2, b4b2 = b4a.reshape(1, _H), b4b.reshape(1, _H)
    bd12, bd22 = bd1.reshape(1, _H), bd2.reshape(1, _OUT)

    degp = deg_sum(ones_h, src, dst, zeros_h)
    x1 = _tc_call(_tc1_body, jax.ShapeDtypeStruct((_N, _H), jnp.float32),
                  degp, W1a, b1a2, W1b, b1b2)

    p2 = seg_sum(x1, src, dst, zeros_h)
    x2m, km = _tc_call(
        _tc2_body,
        (jax.ShapeDtypeStruct((_N, _H), jnp.float32),
         jax.ShapeDtypeStruct((_N, 1), jnp.float32)),
        x1, p2, batch2, w_pool, W2a, b2a2, W2b, b2b2)

    p3 = seg_sum(x2m, src, dst, zeros_h)
    x3m = _tc_call(_tc3_body, jax.ShapeDtypeStruct((_N, _H), jnp.float32),
                   x2m, p3, km, W3a, b3a2, W3b, b3b2)

    p4 = seg_sum(x3m, src, dst, zeros_h)
    out = _tc_call(_tc4_body, jax.ShapeDtypeStruct((_G, _OUT), jnp.float32),
                   x3m, p4, km, batch2, W4a, b4a2, W4b, b4b2,
                   Wd1, bd12, Wd2, bd22)
    return out


# speedup vs baseline: 19.8707x; 1.7001x over previous
import functools

import jax
import jax.numpy as jnp
from jax import lax
from jax.experimental import pallas as pl
from jax.experimental.pallas import tpu as pltpu
from jax.experimental.pallas import tpu_sc as plsc

_N = 10000
_E = 320000
_G = 64
_H = 128
_OUT = 128

_NC = 2
_NS = 16
_NW = _NC * _NS
_CH = 128
_NCHUNK = _E // _CH
_RB = 200
_NBLK = _N // _RB


def _seg_sum_body(width, gather, x_hbm, src_hbm, dst_hbm, zero_hbm, out_hbm,
                  src_v0, src_v1, dst_v0, dst_v1, rows_v0, rows_v1, acc,
                  gsem0, gsem1, ssem0, ssem1, dsem0, dsem1):
    cid = lax.axis_index("c")
    sid = lax.axis_index("s")
    wid = sid * _NC + cid
    srcs = (src_v0, src_v1)
    dsts = (dst_v0, dst_v1)
    rows = (rows_v0, rows_v1)
    gsems = (gsem0, gsem1)
    ssems = (ssem0, ssem1)
    dsems = (dsem0, dsem1)

    nb = (_NBLK - sid + _NS - 1) // _NS

    def init_step(k, carry):
        base = (sid + k * _NS) * _RB
        pltpu.sync_copy(zero_hbm.at[pl.ds(base, _RB)],
                        acc.at[pl.ds(base, _RB)])
        return carry

    lax.fori_loop(0, nb, init_step, 0)
    if not gather:
        pltpu.sync_copy(x_hbm.at[pl.ds(0, _CH)], rows_v0)
    plsc.subcore_barrier()

    nt = (_NCHUNK - wid + _NW - 1) // _NW

    def cbase(t):
        return (wid + t * _NW) * _CH

    for b in range(2):
        pltpu.async_copy(dst_hbm.at[pl.ds(cbase(b), _CH)], dsts[b], dsems[b])
        if gather:
            pltpu.async_copy(src_hbm.at[pl.ds(cbase(b), _CH)], srcs[b],
                             ssems[b])
    if gather:
        pltpu.make_async_copy(src_hbm.at[pl.ds(cbase(0), _CH)], srcs[0],
                              ssems[0]).wait()
        pltpu.async_copy(x_hbm.at[srcs[0]], rows[0], gsems[0])

    def pair(j, carry):
        for b in range(2):
            t = 2 * j + b

            @pl.when(t < nt)
            def _step():
                if gather:
                    @pl.when(t + 1 < nt)
                    def _issue_next():
                        pltpu.make_async_copy(
                            src_hbm.at[pl.ds(cbase(t + 1), _CH)],
                            srcs[1 - b], ssems[1 - b]).wait()
                        pltpu.async_copy(x_hbm.at[srcs[1 - b]], rows[1 - b],
                                         gsems[1 - b])

                    pltpu.make_async_copy(x_hbm.at[srcs[b]], rows[b],
                                          gsems[b]).wait()
                pltpu.make_async_copy(dst_hbm.at[pl.ds(cbase(t), _CH)],
                                      dsts[b], dsems[b]).wait()
                pltpu.sync_copy(rows[b] if gather else rows_v0,
                                acc.at[dsts[b]], add=True)

                @pl.when(t + 2 < nt)
                def _prefetch():
                    pltpu.async_copy(dst_hbm.at[pl.ds(cbase(t + 2), _CH)],
                                     dsts[b], dsems[b])
                    if gather:
                        pltpu.async_copy(src_hbm.at[pl.ds(cbase(t + 2), _CH)],
                                         srcs[b], ssems[b])
        return carry

    lax.fori_loop(0, (nt + 1) // 2, pair, 0)
    plsc.subcore_barrier()

    def wb_step(k, carry):
        base = (sid + k * _NS) * _RB
        pltpu.sync_copy(acc.at[pl.ds(base, _RB)],
                        out_hbm.at[cid, pl.ds(base, _RB)])
        return carry

    lax.fori_loop(0, nb, wb_step, 0)


def _make_seg_sum(width, gather):
    mesh = plsc.VectorSubcoreMesh(core_axis_name="c", subcore_axis_name="s")
    return pl.kernel(
        functools.partial(_seg_sum_body, width, gather),
        out_type=jax.ShapeDtypeStruct((_NC, _N, width), jnp.float32),
        mesh=mesh,
        scratch_types=[
            pltpu.VMEM((_CH,), jnp.int32),
            pltpu.VMEM((_CH,), jnp.int32),
            pltpu.VMEM((_CH,), jnp.int32),
            pltpu.VMEM((_CH,), jnp.int32),
            pltpu.VMEM((_CH, width), jnp.float32),
            pltpu.VMEM((_CH, width), jnp.float32),
            pltpu.VMEM_SHARED((_N, width), jnp.float32),
            pltpu.SemaphoreType.DMA,
            pltpu.SemaphoreType.DMA,
            pltpu.SemaphoreType.DMA,
            pltpu.SemaphoreType.DMA,
            pltpu.SemaphoreType.DMA,
            pltpu.SemaphoreType.DMA,
        ],
        name=f"sc_seg_sum_{width}_{int(gather)}",
    )


def _relu(x):
    return jnp.maximum(x, 0.0)


def _mm(a, b):
    return jax.lax.dot_general(a, b, (((1,), (0,)), ((), ())),
                               preferred_element_type=jnp.float32)


def _tc1_body(degp_ref, w1a_ref, b1a_ref, w1b_ref, b1b_ref, x1_ref):
    deg = degp_ref[0][:, :1] + degp_ref[1][:, :1]
    t = deg + 1.0
    h = _relu(t * w1a_ref[...] + b1a_ref[...])
    x1_ref[...] = _relu(_mm(h, w1b_ref[...]) + b1b_ref[...])


def _tc2_body(x1_ref, p_ref, batch_ref, wp_ref,
              w2a_ref, b2a_ref, w2b_ref, b2b_ref, x2m_ref, km_ref):
    h0 = x1_ref[...] + p_ref[0] + p_ref[1]
    h1 = _relu(_mm(h0, w2a_ref[...]) + b2a_ref[...])
    x2 = _relu(_mm(h1, w2b_ref[...]) + b2b_ref[...])
    s = jnp.sum(x2, axis=1, keepdims=True)
    wp = wp_ref[0, 0]
    score = jnp.tanh(s * (wp / jnp.abs(wp)))

    bat = batch_ref[...]
    gid = lax.broadcasted_iota(jnp.int32, (1, _G), 1)
    onehot = bat == gid
    counts = jnp.sum(onehot.astype(jnp.float32), axis=0, keepdims=True)
    kk = jnp.ceil(0.5 * counts)

    key = bat.astype(jnp.float32) * 4.0 - score
    b = jax.lax.bitcast_convert_type(key, jnp.uint32)
    neg = (b >> jnp.uint32(31)) > jnp.uint32(0)
    u = jnp.where(neg, ~b, b | jnp.uint32(0x80000000))
    v = ~u

    def bs_step(i, thr):
        bit = jnp.uint32(1) << (jnp.uint32(31) - i.astype(jnp.uint32))
        cand = thr | bit
        ge = jnp.logical_and(v >= cand, onehot)
        cnt = jnp.sum(ge.astype(jnp.float32), axis=0, keepdims=True)
        return jnp.where(cnt >= kk, cand, thr)

    thr = lax.fori_loop(0, 32, bs_step, jnp.zeros((1, _G), jnp.uint32))
    gt_ng = jnp.logical_and(v > thr, onehot)
    cnt_gt = jnp.sum(gt_ng.astype(jnp.float32), axis=0, keepdims=True)
    rr = kk - cnt_gt
    bnd = jnp.logical_and(v == thr, onehot)
    idx = lax.broadcasted_iota(jnp.uint32, (_N, 1), 0)
    w = jnp.uint32(_N) - idx

    def bs_step2(i, thr2):
        bit = jnp.uint32(1) << (jnp.uint32(15) - i.astype(jnp.uint32))
        cand = thr2 | bit
        ge = jnp.logical_and(w >= cand, bnd)
        cnt = jnp.sum(ge.astype(jnp.float32), axis=0, keepdims=True)
        return jnp.where(cnt >= rr, cand, thr2)

    thr2 = lax.fori_loop(0, 16, bs_step2, jnp.zeros((1, _G), jnp.uint32))
    keep_ng = jnp.logical_or(gt_ng, jnp.logical_and(bnd, w >= thr2))
    km = jnp.sum(keep_ng.astype(jnp.float32), axis=1, keepdims=True)
    km_ref[...] = km
    x2m_ref[...] = x2 * (score * km)


def _tc3_body(xm_ref, p_ref, km_ref, wa_ref, ba_ref, wb_ref, bb_ref, o_ref):
    h0 = xm_ref[...] + p_ref[0] + p_ref[1]
    h1 = _relu(_mm(h0, wa_ref[...]) + ba_ref[...])
    o_ref[...] = _relu(_mm(h1, wb_ref[...]) + bb_ref[...]) * km_ref[...]


def _tc4_body(xm_ref, p_ref, km_ref, batch_ref,
              w4a_ref, b4a_ref, w4b_ref, b4b_ref,
              wd1_ref, bd1_ref, wd2_ref, bd2_ref, out_ref):
    h0 = xm_ref[...] + p_ref[0] + p_ref[1]
    h1 = _relu(_mm(h0, w4a_ref[...]) + b4a_ref[...])
    x4 = _relu(_mm(h1, w4b_ref[...]) + b4b_ref[...]) * km_ref[...]
    gid = lax.broadcasted_iota(jnp.int32, (1, _G), 1)
    onehot = (batch_ref[...] == gid).astype(jnp.float32)
    pooled = jax.lax.dot_general(onehot, x4, (((0,), (0,)), ((), ())),
                                 preferred_element_type=jnp.float32)
    hh = _relu(_mm(pooled, wd1_ref[...]) + bd1_ref[...])
    out_ref[...] = _mm(hh, wd2_ref[...]) + bd2_ref[...]


def _tc_call(body, out_shape, *args):
    return pl.pallas_call(
        body, out_shape=out_shape, name=body.__name__)(*args)


def kernel(edge_index, batch, W1a, b1a, W1b, b1b, W2a, b2a, W2b, b2b,
           W3a, b3a, W3b, b3b, W4a, b4a, W4b, b4b, Wd1, bd1, Wd2, bd2,
           w_pool):
    src = edge_index[0]
    dst = edge_index[1]
    batch2 = batch.reshape(_N, 1)
    zeros_h = jnp.zeros((_N, _H), jnp.float32)
    ones_h = jnp.ones((_CH, _H), jnp.float32)

    deg_sum = _make_seg_sum(_H, gather=False)
    seg_sum = _make_seg_sum(_H, gather=True)

    b1a2, b1b2 = b1a.reshape(1, _H), b1b.reshape(1, _H)
    b2a2, b2b2 = b2a.reshape(1, _H), b2b.reshape(1, _H)
    b3a2, b3b2 = b3a.reshape(1, _H), b3b.reshape(1, _H)
    b4a2, b4b2 = b4a.reshape(1, _H), b4b.reshape(1, _H)
    bd12, bd22 = bd1.reshape(1, _H), bd2.reshape(1, _OUT)

    degp = deg_sum(ones_h, src, dst, zeros_h)
    x1 = _tc_call(_tc1_body, jax.ShapeDtypeStruct((_N, _H), jnp.float32),
                  degp, W1a, b1a2, W1b, b1b2)

    p2 = seg_sum(x1, src, dst, zeros_h)
    x2m, km = _tc_call(
        _tc2_body,
        (jax.ShapeDtypeStruct((_N, _H), jnp.float32),
         jax.ShapeDtypeStruct((_N, 1), jnp.float32)),
        x1, p2, batch2, w_pool, W2a, b2a2, W2b, b2b2)

    p3 = seg_sum(x2m, src, dst, zeros_h)
    x3m = _tc_call(_tc3_body, jax.ShapeDtypeStruct((_N, _H), jnp.float32),
                   x2m, p3, km, W3a, b3a2, W3b, b3b2)

    p4 = seg_sum(x3m, src, dst, zeros_h)
    out = _tc_call(_tc4_body, jax.ShapeDtypeStruct((_G, _OUT), jnp.float32),
                   x3m, p4, km, batch2, W4a, b4a2, W4b, b4b2,
                   Wd1, bd12, Wd2, bd22)
    return out


# speedup vs baseline: 22.4555x; 1.1301x over previous
import functools

import jax
import jax.numpy as jnp
from jax import lax
from jax.experimental import pallas as pl
from jax.experimental.pallas import tpu as pltpu
from jax.experimental.pallas import tpu_sc as plsc

_N = 10000
_E = 320000
_G = 64
_H = 128
_OUT = 128

_NC = 2
_NS = 16
_NW = _NC * _NS
_CH = 128
_NCHUNK = _E // _CH
_RB = 200
_NBLK = _N // _RB
_NP = 10112


def _seg_sum_body(x_hbm, edge_hbm, zero_hbm, out_hbm,
                  src_v0, src_v1, dst_v0, dst_v1, rows_v0, rows_v1, acc,
                  gsem0, gsem1, ssem0, ssem1, dsem0, dsem1):
    cid = lax.axis_index("c")
    sid = lax.axis_index("s")
    wid = sid * _NC + cid
    srcs = (src_v0, src_v1)
    dsts = (dst_v0, dst_v1)
    rows = (rows_v0, rows_v1)
    gsems = (gsem0, gsem1)
    ssems = (ssem0, ssem1)
    dsems = (dsem0, dsem1)

    nb = (_NBLK - sid + _NS - 1) // _NS

    def init_step(k, carry):
        base = (sid + k * _NS) * _RB
        pltpu.sync_copy(zero_hbm.at[pl.ds(base, _RB)],
                        acc.at[pl.ds(base, _RB)])
        return carry

    lax.fori_loop(0, nb, init_step, 0)
    plsc.subcore_barrier()

    nt = (_NCHUNK - wid + _NW - 1) // _NW

    def cbase(t):
        return (wid + t * _NW) * _CH

    for b in range(2):
        pltpu.async_copy(edge_hbm.at[1, pl.ds(cbase(b), _CH)], dsts[b],
                         dsems[b])
        pltpu.async_copy(edge_hbm.at[0, pl.ds(cbase(b), _CH)], srcs[b],
                         ssems[b])
    pltpu.make_async_copy(edge_hbm.at[0, pl.ds(cbase(0), _CH)], srcs[0],
                          ssems[0]).wait()
    pltpu.async_copy(x_hbm.at[srcs[0]], rows[0], gsems[0])

    def pair(j, carry):
        for b in range(2):
            t = 2 * j + b

            @pl.when(t < nt)
            def _step():
                @pl.when(t + 1 < nt)
                def _issue_next():
                    pltpu.make_async_copy(
                        edge_hbm.at[0, pl.ds(cbase(t + 1), _CH)],
                        srcs[1 - b], ssems[1 - b]).wait()
                    pltpu.async_copy(x_hbm.at[srcs[1 - b]], rows[1 - b],
                                     gsems[1 - b])

                pltpu.make_async_copy(x_hbm.at[srcs[b]], rows[b],
                                      gsems[b]).wait()
                pltpu.make_async_copy(edge_hbm.at[1, pl.ds(cbase(t), _CH)],
                                      dsts[b], dsems[b]).wait()
                pltpu.sync_copy(rows[b], acc.at[dsts[b]], add=True)

                @pl.when(t + 2 < nt)
                def _prefetch():
                    pltpu.async_copy(edge_hbm.at[1, pl.ds(cbase(t + 2), _CH)],
                                     dsts[b], dsems[b])
                    pltpu.async_copy(edge_hbm.at[0, pl.ds(cbase(t + 2), _CH)],
                                     srcs[b], ssems[b])
        return carry

    lax.fori_loop(0, (nt + 1) // 2, pair, 0)
    plsc.subcore_barrier()

    def wb_step(k, carry):
        base = (sid + k * _NS) * _RB
        pltpu.sync_copy(acc.at[pl.ds(base, _RB)],
                        out_hbm.at[cid, pl.ds(base, _RB)])
        return carry

    lax.fori_loop(0, nb, wb_step, 0)


def _make_seg_sum():
    mesh = plsc.VectorSubcoreMesh(core_axis_name="c", subcore_axis_name="s")
    return pl.kernel(
        _seg_sum_body,
        out_type=jax.ShapeDtypeStruct((_NC, _N, _H), jnp.float32),
        mesh=mesh,
        scratch_types=[
            pltpu.VMEM((_CH,), jnp.int32),
            pltpu.VMEM((_CH,), jnp.int32),
            pltpu.VMEM((_CH,), jnp.int32),
            pltpu.VMEM((_CH,), jnp.int32),
            pltpu.VMEM((_CH, _H), jnp.float32),
            pltpu.VMEM((_CH, _H), jnp.float32),
            pltpu.VMEM_SHARED((_N, _H), jnp.float32),
            pltpu.SemaphoreType.DMA,
            pltpu.SemaphoreType.DMA,
            pltpu.SemaphoreType.DMA,
            pltpu.SemaphoreType.DMA,
            pltpu.SemaphoreType.DMA,
            pltpu.SemaphoreType.DMA,
        ],
        name="sc_seg_sum",
    )


def _deg_body(edge_hbm, zero_hbm, out_hbm,
              dst_v0, dst_v1, deg_v, dsem0, dsem1):
    cid = lax.axis_index("c")
    sid = lax.axis_index("s")
    wid = sid * _NC + cid
    dsts = (dst_v0, dst_v1)
    dsems = (dsem0, dsem1)

    pltpu.sync_copy(zero_hbm, deg_v)

    nt = (_NCHUNK - wid + _NW - 1) // _NW

    def cbase(t):
        return (wid + t * _NW) * _CH

    for b in range(2):
        pltpu.async_copy(edge_hbm.at[1, pl.ds(cbase(b), _CH)], dsts[b],
                         dsems[b])

    ones = jnp.ones((16,), jnp.float32)

    def pair(j, carry):
        for b in range(2):
            t = 2 * j + b

            @pl.when(t < nt)
            def _step():
                pltpu.make_async_copy(edge_hbm.at[1, pl.ds(cbase(t), _CH)],
                                      dsts[b], dsems[b]).wait()
                for k in range(_CH // 16):
                    iv = dsts[b][pl.ds(k * 16, 16)]
                    plsc.addupdate_scatter(deg_v, [iv], ones)

                @pl.when(t + 2 < nt)
                def _prefetch():
                    pltpu.async_copy(edge_hbm.at[1, pl.ds(cbase(t + 2), _CH)],
                                     dsts[b], dsems[b])
        return carry

    lax.fori_loop(0, (nt + 1) // 2, pair, 0)
    pltpu.sync_copy(deg_v, out_hbm.at[wid])


def _make_deg():
    mesh = plsc.VectorSubcoreMesh(core_axis_name="c", subcore_axis_name="s")
    return pl.kernel(
        _deg_body,
        out_type=jax.ShapeDtypeStruct((_NW, _NP), jnp.float32),
        mesh=mesh,
        scratch_types=[
            pltpu.VMEM((_CH,), jnp.int32),
            pltpu.VMEM((_CH,), jnp.int32),
            pltpu.VMEM((_NP,), jnp.float32),
            pltpu.SemaphoreType.DMA,
            pltpu.SemaphoreType.DMA,
        ],
        compiler_params=pltpu.CompilerParams(needs_layout_passes=False),
        name="sc_deg",
    )


def _relu(x):
    return jnp.maximum(x, 0.0)


def _mm(a, b):
    return jax.lax.dot_general(a, b, (((1,), (0,)), ((), ())),
                               preferred_element_type=jnp.float32)


def _outer(a, b):
    return jax.lax.dot_general(a, b, (((0,), (0,)), ((), ())),
                               preferred_element_type=jnp.float32)


def _onehot_from_batch(batch_1n):
    gidr = lax.broadcasted_iota(jnp.int32, (_G, 1), 0)
    cmp = (batch_1n == gidr).astype(jnp.float32)
    cnt_g1 = jnp.sum(cmp, axis=1, keepdims=True)
    cnt = jnp.transpose(cnt_g1)
    ii = lax.broadcasted_iota(jnp.int32, (_G, _G), 0)
    jj = lax.broadcasted_iota(jnp.int32, (_G, _G), 1)
    ltri = (ii <= jj).astype(jnp.float32)
    cum = _mm(cnt, ltri)
    off = cum - cnt
    ni = lax.broadcasted_iota(jnp.int32, (_N, 1), 0).astype(jnp.float32)
    onehot = jnp.logical_and(ni >= off, ni < cum)
    return onehot, cnt


def _tc1_body(degp_ref, w1a_ref, b1a_ref, w1b_ref, b1b_ref, x1_ref):
    degs = jnp.sum(degp_ref[...], axis=0, keepdims=True)
    t = degs[:, :_N] + 1.0
    h = _relu(_outer(t, w1a_ref[...]) + b1a_ref[...])
    x1_ref[...] = _relu(_mm(h, w1b_ref[...]) + b1b_ref[...])


def _tc2_body(x1_ref, p_ref, batch_ref, wp_ref,
              w2a_ref, b2a_ref, w2b_ref, b2b_ref, x2m_ref, km_ref):
    h0 = x1_ref[...] + p_ref[0] + p_ref[1]
    h1 = _relu(_mm(h0, w2a_ref[...]) + b2a_ref[...])
    x2 = _relu(_mm(h1, w2b_ref[...]) + b2b_ref[...])
    s = jnp.sum(x2, axis=1, keepdims=True)
    wp = wp_ref[0, 0]
    score = jnp.tanh(s * (wp / jnp.abs(wp)))

    onehot, cnt = _onehot_from_batch(batch_ref[...])
    kk = jnp.ceil(0.5 * cnt)

    gidf = lax.broadcasted_iota(jnp.int32, (1, _G), 1).astype(jnp.float32)
    gids = jnp.sum(onehot.astype(jnp.float32) * gidf,
                   axis=1, keepdims=True)
    key = gids * 4.0 - score
    b = jax.lax.bitcast_convert_type(key, jnp.uint32)
    neg = (b >> jnp.uint32(31)) > jnp.uint32(0)
    u = jnp.where(neg, ~b, b | jnp.uint32(0x80000000))
    v = ~u

    def bs_step(i, thr):
        bit = jnp.uint32(1) << (jnp.uint32(31) - i.astype(jnp.uint32))
        cand = thr | bit
        ge = jnp.logical_and(v >= cand, onehot)
        cntc = jnp.sum(ge.astype(jnp.float32), axis=0, keepdims=True)
        return jnp.where(cntc >= kk, cand, thr)

    thr = lax.fori_loop(0, 32, bs_step, jnp.zeros((1, _G), jnp.uint32))
    gt_ng = jnp.logical_and(v > thr, onehot)
    cnt_gt = jnp.sum(gt_ng.astype(jnp.float32), axis=0, keepdims=True)
    rr = kk - cnt_gt
    bnd = jnp.logical_and(v == thr, onehot)
    idx = lax.broadcasted_iota(jnp.uint32, (_N, 1), 0)
    w = jnp.uint32(_N) - idx

    def bs_step2(i, thr2):
        bit = jnp.uint32(1) << (jnp.uint32(15) - i.astype(jnp.uint32))
        cand = thr2 | bit
        ge = jnp.logical_and(w >= cand, bnd)
        cntc = jnp.sum(ge.astype(jnp.float32), axis=0, keepdims=True)
        return jnp.where(cntc >= rr, cand, thr2)

    thr2 = lax.fori_loop(0, 16, bs_step2, jnp.zeros((1, _G), jnp.uint32))
    keep_ng = jnp.logical_or(gt_ng, jnp.logical_and(bnd, w >= thr2))
    km = jnp.sum(keep_ng.astype(jnp.float32), axis=1, keepdims=True)
    km_ref[...] = km
    x2m_ref[...] = x2 * (score * km)


def _tc3_body(xm_ref, p_ref, km_ref, wa_ref, ba_ref, wb_ref, bb_ref, o_ref):
    h0 = xm_ref[...] + p_ref[0] + p_ref[1]
    h1 = _relu(_mm(h0, wa_ref[...]) + ba_ref[...])
    o_ref[...] = _relu(_mm(h1, wb_ref[...]) + bb_ref[...]) * km_ref[...]


def _tc4_body(xm_ref, p_ref, km_ref, batch_ref,
              w4a_ref, b4a_ref, w4b_ref, b4b_ref,
              wd1_ref, bd1_ref, wd2_ref, bd2_ref, out_ref):
    h0 = xm_ref[...] + p_ref[0] + p_ref[1]
    h1 = _relu(_mm(h0, w4a_ref[...]) + b4a_ref[...])
    x4 = _relu(_mm(h1, w4b_ref[...]) + b4b_ref[...]) * km_ref[...]
    onehot, _ = _onehot_from_batch(batch_ref[...])
    pooled = jax.lax.dot_general(onehot.astype(jnp.float32), x4,
                                 (((0,), (0,)), ((), ())),
                                 preferred_element_type=jnp.float32)
    hh = _relu(_mm(pooled, wd1_ref[...]) + bd1_ref[...])
    out_ref[...] = _mm(hh, wd2_ref[...]) + bd2_ref[...]


def _tc_call(body, out_shape, *args):
    return pl.pallas_call(
        body, out_shape=out_shape, name=body.__name__)(*args)


def kernel(edge_index, batch, W1a, b1a, W1b, b1b, W2a, b2a, W2b, b2b,
           W3a, b3a, W3b, b3b, W4a, b4a, W4b, b4b, Wd1, bd1, Wd2, bd2,
           w_pool):
    batch1n = batch.reshape(1, _N)
    zeros_h = jnp.zeros((_N, _H), jnp.float32)
    zeros_np = jnp.zeros((_NP,), jnp.float32)

    deg_sum = _make_deg()
    seg_sum = _make_seg_sum()

    b1a2, b1b2 = b1a.reshape(1, _H), b1b.reshape(1, _H)
    b2a2, b2b2 = b2a.reshape(1, _H), b2b.reshape(1, _H)
    b3a2, b3b2 = b3a.reshape(1, _H), b3b.reshape(1, _H)
    b4a2, b4b2 = b4a.reshape(1, _H), b4b.reshape(1, _H)
    bd12, bd22 = bd1.reshape(1, _H), bd2.reshape(1, _OUT)

    degp = deg_sum(edge_index, zeros_np)
    x1 = _tc_call(_tc1_body, jax.ShapeDtypeStruct((_N, _H), jnp.float32),
                  degp, W1a, b1a2, W1b, b1b2)

    p2 = seg_sum(x1, edge_index, zeros_h)
    x2m, km = _tc_call(
        _tc2_body,
        (jax.ShapeDtypeStruct((_N, _H), jnp.float32),
         jax.ShapeDtypeStruct((_N, 1), jnp.float32)),
        x1, p2, batch1n, w_pool, W2a, b2a2, W2b, b2b2)

    p3 = seg_sum(x2m, edge_index, zeros_h)
    x3m = _tc_call(_tc3_body, jax.ShapeDtypeStruct((_N, _H), jnp.float32),
                   x2m, p3, km, W3a, b3a2, W3b, b3b2)

    p4 = seg_sum(x3m, edge_index, zeros_h)
    out = _tc_call(_tc4_body, jax.ShapeDtypeStruct((_G, _OUT), jnp.float32),
                   x3m, p4, km, batch1n, W4a, b4a2, W4b, b4b2,
                   Wd1, bd12, Wd2, bd22)
    return out
